# baseline (speedup 1.0000x reference)
import jax
import jax.numpy as jnp
from jax import lax
from jax.experimental import pallas as pl
from jax.experimental.pallas import tpu as pltpu


def _disp_conv_kernel(wmix_ref, b_ref, x_ref, halo_ref, out_ref,
                      xt_ref, d_ref, m_ref):
    _, c, tile_h, w = x_ref.shape
    th2 = tile_h + 2
    output_nc = out_ref.shape[1]
    max_shift = d_ref.shape[1]

    xt_ref[:, 0:w, :] = jnp.transpose(x_ref[0], (0, 2, 1))
    xt_ref[:, w:w + max_shift, :] = jnp.transpose(
        x_ref[0, :, :, 0:max_shift], (0, 2, 1))

    n_blk = d_ref.shape[0]
    starts = [min(8 * t, th2 - 8) for t in range(n_blk)]

    unroll = 2
    while max_shift % unroll:
        unroll //= 2

    def cv_step(cg, carry):
        for u in range(unroll):
            ci = cg * unroll + u
            s = ci + 1
            base = xt_ref[:, 0:w, :]
            shft = xt_ref[:, pl.ds(s, w), :]
            dt = jnp.sum(jnp.abs(base - shft), axis=0)
            dtt = jnp.transpose(dt, (1, 0))
            for t, i0 in enumerate(starts):
                lo = max(i0, 1)
                hi = min(i0 + 8, th2 - 1)
                d_ref[t, pl.ds(ci, 1), lo - i0:hi - i0, :] = (
                    dtt[lo - 1:hi - 1][None])
        return carry

    lax.fori_loop(0, max_shift // unroll, cv_step, 0)

    def cv_halo_step(ci, carry):
        s = ci + 1
        amt = (w - s) % w
        hb = halo_ref[0, 0]
        dh = jnp.sum(jnp.abs(hb - pltpu.roll(hb, shift=amt, axis=2)),
                     axis=0)
        d_ref[0, pl.ds(ci, 1), 0:1, :] = dh[0:1][None]
        d_ref[n_blk - 1, pl.ds(ci, 1), 7:8, :] = dh[1:2][None]
        return carry

    lax.fori_loop(0, max_shift, cv_halo_step, 0)

    w8 = wmix_ref[...]
    for t, i0 in enumerate(starts):
        dv = d_ref[t].reshape(max_shift * 8, w)
        m = jnp.dot(w8, dv, preferred_element_type=jnp.float32)
        m_ref[:, i0:i0 + 8, :] = m.reshape(m.shape[0] // 8, 8, w)

    col = lax.broadcasted_iota(jnp.int32, (tile_h, w), 1)
    okl = col > 0
    okr = col < w - 1
    for o in range(output_nc):
        acc = jnp.full((tile_h, w), b_ref[o], jnp.float32)
        for ki in range(3):
            for kj in range(3):
                q = o * 9 + ki * 3 + kj
                t = m_ref[q, ki:ki + tile_h, :]
                if kj == 0:
                    t = jnp.where(okl, pltpu.roll(t, shift=1, axis=1), 0.0)
                elif kj == 2:
                    t = jnp.where(okr, pltpu.roll(t, shift=w - 1, axis=1), 0.0)
                acc = acc + t
        out_ref[0, o] = acc.astype(out_ref.dtype)


def _pick_tile_h(h):
    for cand in (128, 64, 32, 16, 8):
        if h % cand == 0:
            return cand
    return h


def kernel(x, weight, bias):
    b, c, h, w = x.shape
    output_nc, max_shift = weight.shape[0], weight.shape[1]

    tile_h = _pick_tile_h(h)
    n_h = h // tile_h

    zero_row = jnp.zeros((b, c, 1, w), x.dtype)
    above = jnp.concatenate([zero_row, x[:, :, tile_h - 1:h - 1:tile_h, :]], axis=2)
    below = jnp.concatenate([x[:, :, tile_h:h:tile_h, :], zero_row], axis=2)
    halo = jnp.transpose(jnp.stack([above, below], axis=3), (0, 2, 1, 3, 4))

    wmix = (weight.astype(jnp.float32) / jnp.float32(c))
    wmix = wmix.reshape(output_nc, max_shift, 9).transpose(1, 0, 2)
    wmix = wmix.reshape(max_shift, output_nc * 9)
    w8 = (wmix.T[:, None, :, None] * jnp.eye(8, dtype=jnp.float32)[None, :, None, :])
    w8 = w8.reshape(output_nc * 9 * 8, max_shift * 8)
    b_flat = bias.astype(jnp.float32)

    return pl.pallas_call(
        _disp_conv_kernel,
        out_shape=jax.ShapeDtypeStruct((b, output_nc, h, w), x.dtype),
        grid=(b, n_h),
        in_specs=[
            pl.BlockSpec((output_nc * 9 * 8, max_shift * 8),
                         lambda bi, hi: (0, 0)),
            pl.BlockSpec(memory_space=pltpu.MemorySpace.SMEM),
            pl.BlockSpec((1, c, tile_h, w), lambda bi, hi: (bi, 0, hi, 0)),
            pl.BlockSpec((1, 1, c, 2, w), lambda bi, hi: (bi, hi, 0, 0, 0)),
        ],
        out_specs=pl.BlockSpec((1, output_nc, tile_h, w),
                               lambda bi, hi: (bi, 0, hi, 0)),
        scratch_shapes=[
            pltpu.VMEM((c, w + max_shift, tile_h), jnp.float32),
            pltpu.VMEM((-(-(tile_h + 2) // 8), max_shift, 8, w), jnp.float32),
            pltpu.VMEM((output_nc * 9, tile_h + 2, w), jnp.float32),
        ],
        compiler_params=pltpu.CompilerParams(
            dimension_semantics=("parallel", "parallel")),
    )(w8, b_flat, x, halo)


# speedup vs baseline: 1.0707x; 1.0707x over previous
import jax
import jax.numpy as jnp
from jax import lax
from jax.experimental import pallas as pl
from jax.experimental.pallas import tpu as pltpu


def _disp_conv_kernel(wmix_ref, b_ref, x_ref, halo_ref, out_ref,
                      xt_ref, d_ref, m_ref):
    _, c, tile_h, w = x_ref.shape
    th2 = tile_h + 2
    output_nc = out_ref.shape[1]
    max_shift = d_ref.shape[1]

    xt_ref[:, 0:w, :] = jnp.transpose(x_ref[0], (0, 2, 1))
    xt_ref[:, w:w + max_shift, :] = jnp.transpose(
        x_ref[0, :, :, 0:max_shift], (0, 2, 1))

    n_blk = d_ref.shape[0]
    starts = [min(8 * t, th2 - 8) for t in range(n_blk)]

    unroll = 2
    while max_shift % unroll:
        unroll //= 2

    def cv_step(cg, carry):
        for u in range(unroll):
            ci = cg * unroll + u
            s = ci + 1
            base = xt_ref[:, 0:w, :]
            shft = xt_ref[:, pl.ds(s, w), :]
            dt = jnp.sum(jnp.abs(base - shft), axis=0)
            dtt = jnp.transpose(dt, (1, 0))
            amt = (w - s) % w
            hb = halo_ref[0, 0]
            dh = jnp.sum(jnp.abs(hb - pltpu.roll(hb, shift=amt, axis=2)),
                         axis=0)
            for t, i0 in enumerate(starts):
                lo = max(i0, 1)
                hi = min(i0 + 8, th2 - 1)
                d_ref[t, pl.ds(ci, 1), lo - i0:hi - i0, :] = (
                    dtt[lo - 1:hi - 1][None])
                if i0 == 0:
                    d_ref[t, pl.ds(ci, 1), 0:1, :] = dh[0:1][None]
                if i0 + 8 == th2:
                    d_ref[t, pl.ds(ci, 1), 7:8, :] = dh[1:2][None]
        return carry

    lax.fori_loop(0, max_shift // unroll, cv_step, 0)

    w8 = wmix_ref[...]
    for t, i0 in enumerate(starts):
        dv = d_ref[t].reshape(max_shift * 8, w)
        m = jnp.dot(w8, dv, preferred_element_type=jnp.float32)
        m_ref[:, i0:i0 + 8, :] = m.reshape(m.shape[0] // 8, 8, w)

    col = lax.broadcasted_iota(jnp.int32, (tile_h, w), 1)
    okl = col > 0
    okr = col < w - 1
    for o in range(output_nc):
        acc = jnp.full((tile_h, w), b_ref[o], jnp.float32)
        for ki in range(3):
            for kj in range(3):
                q = o * 9 + ki * 3 + kj
                t = m_ref[q, ki:ki + tile_h, :]
                if kj == 0:
                    t = jnp.where(okl, pltpu.roll(t, shift=1, axis=1), 0.0)
                elif kj == 2:
                    t = jnp.where(okr, pltpu.roll(t, shift=w - 1, axis=1), 0.0)
                acc = acc + t
        out_ref[0, o] = acc.astype(out_ref.dtype)


def _pick_tile_h(h):
    for cand in (128, 64, 32, 16, 8):
        if h % cand == 0:
            return cand
    return h


def kernel(x, weight, bias):
    b, c, h, w = x.shape
    output_nc, max_shift = weight.shape[0], weight.shape[1]

    tile_h = _pick_tile_h(h)
    n_h = h // tile_h

    zero_row = jnp.zeros((b, c, 1, w), x.dtype)
    above = jnp.concatenate([zero_row, x[:, :, tile_h - 1:h - 1:tile_h, :]], axis=2)
    below = jnp.concatenate([x[:, :, tile_h:h:tile_h, :], zero_row], axis=2)
    halo = jnp.transpose(jnp.stack([above, below], axis=3), (0, 2, 1, 3, 4))

    wmix = (weight.astype(jnp.float32) / jnp.float32(c))
    wmix = wmix.reshape(output_nc, max_shift, 9).transpose(1, 0, 2)
    wmix = wmix.reshape(max_shift, output_nc * 9)
    w8 = (wmix.T[:, None, :, None] * jnp.eye(8, dtype=jnp.float32)[None, :, None, :])
    w8 = w8.reshape(output_nc * 9 * 8, max_shift * 8)
    b_flat = bias.astype(jnp.float32)

    return pl.pallas_call(
        _disp_conv_kernel,
        out_shape=jax.ShapeDtypeStruct((b, output_nc, h, w), x.dtype),
        grid=(b, n_h),
        in_specs=[
            pl.BlockSpec((output_nc * 9 * 8, max_shift * 8),
                         lambda bi, hi: (0, 0)),
            pl.BlockSpec(memory_space=pltpu.MemorySpace.SMEM),
            pl.BlockSpec((1, c, tile_h, w), lambda bi, hi: (bi, 0, hi, 0)),
            pl.BlockSpec((1, 1, c, 2, w), lambda bi, hi: (bi, hi, 0, 0, 0)),
        ],
        out_specs=pl.BlockSpec((1, output_nc, tile_h, w),
                               lambda bi, hi: (bi, 0, hi, 0)),
        scratch_shapes=[
            pltpu.VMEM((c, w + max_shift, tile_h), jnp.float32),
            pltpu.VMEM((-(-(tile_h + 2) // 8), max_shift, 8, w), jnp.float32),
            pltpu.VMEM((output_nc * 9, tile_h + 2, w), jnp.float32),
        ],
        compiler_params=pltpu.CompilerParams(
            dimension_semantics=("parallel", "parallel")),
    )(w8, b_flat, x, halo)
